# baseline (speedup 1.0000x reference)
import functools

import jax
import jax.numpy as jnp
from jax import lax
from jax.experimental import pallas as pl
from jax.experimental.pallas import tpu as pltpu
from jax.experimental.pallas import tpu_sc as plsc

NN = 100000
DD = 128
EE = 500000
AA = 100.0
EPSILON = 0.1
NUM_NEG = 5

NC = 2
NS = 16
NW = NC * NS
LL = 16

EP = 512000
K1 = 128
B1 = EP // K1
BPW1 = B1 // NW

EPB = EP // 128

K2 = 800
NB2 = EP // K2
BPT2 = NB2 // NS
SEL = 2 * K2 + 64
MAXB = SEL // 128
RCH = 12544
NCH = 8
RPT = RCH // NS
NP = NCH * RCH

_mesh = plsc.VectorSubcoreMesh(core_axis_name="c", subcore_axis_name="s")


@functools.partial(
    pl.kernel,
    mesh=_mesh,
    compiler_params=pltpu.CompilerParams(needs_layout_passes=False),
    out_type=jax.ShapeDtypeStruct((EP,), jnp.float32),
    scratch_types=[
        pltpu.VMEM((K1,), jnp.int32),
        pltpu.VMEM((K1,), jnp.int32),
        pltpu.VMEM((K1, DD), jnp.float32),
        pltpu.VMEM((K1, DD), jnp.float32),
        pltpu.VMEM((K1,), jnp.float32),
        pltpu.SemaphoreType.DMA,
        pltpu.SemaphoreType.DMA,
    ],
)
def _inner_kernel(x_hbm, u_hbm, v_hbm, out_hbm, uu, vv, xu, xv, innb,
                  semu, semv):
    wid = lax.axis_index("s") * NC + lax.axis_index("c")
    lane = lax.iota(jnp.int32, LL)
    jv = jnp.where(lane == 0, -1.0, 1.0).astype(jnp.float32)

    def block(t, carry):
        base = (wid * BPW1 + t) * K1
        pltpu.sync_copy(u_hbm.at[pl.ds(base, K1)], uu)
        pltpu.sync_copy(v_hbm.at[pl.ds(base, K1)], vv)
        cu = pltpu.async_copy(x_hbm.at[uu], xu, semu)
        cv = pltpu.async_copy(x_hbm.at[vv], xv, semv)
        cu.wait()
        cv.wait()

        def group(g, c):
            innv = jnp.zeros((LL,), jnp.float32)
            for i in range(LL):
                e = g * LL + i
                acc = xu[e, pl.ds(0, LL)] * xv[e, pl.ds(0, LL)] * jv
                for k in range(1, DD // LL):
                    acc = acc + xu[e, pl.ds(k * LL, LL)] * xv[e, pl.ds(k * LL, LL)]
                innv = jnp.where(lane == i, jnp.sum(acc), innv)
            innb[pl.ds(g * LL, LL)] = innv
            return c

        lax.fori_loop(0, K1 // LL, group, 0)
        pltpu.sync_copy(innb, out_hbm.at[pl.ds(base, K1)])
        return carry

    lax.fori_loop(0, BPW1, block, 0)


def _gate_body(inner_ref, factor_ref, energy_ref):
    z = -jnp.minimum(inner_ref[...], -1.0 - 1e-7)
    denom = jnp.sqrt((z - 1.0) * (z + 1.0))
    dist = jnp.log(z + denom)
    row = lax.broadcasted_iota(jnp.int32, (EPB, 128), 0)
    col = lax.broadcasted_iota(jnp.int32, (EPB, 128), 1)
    valid = (row * 128 + col) < EE
    active = (dist < EPSILON) & valid
    delta = jnp.where(active, EPSILON - dist, 0.0)
    factor_ref[...] = -(AA / NUM_NEG) * delta / (denom + 1e-9)
    energy = (0.5 * AA / NUM_NEG) * jnp.sum(delta * delta)
    energy_ref[...] = jnp.broadcast_to(energy, (1, 1))


_gate = pl.pallas_call(
    _gate_body,
    out_shape=(jax.ShapeDtypeStruct((EPB, 128), jnp.float32),
               jax.ShapeDtypeStruct((1, 1), jnp.float32)),
)


@functools.partial(
    pl.kernel,
    mesh=_mesh,
    compiler_params=pltpu.CompilerParams(needs_layout_passes=False),
    out_type=jax.ShapeDtypeStruct((NP, DD), jnp.float32),
    scratch_types=[
        pltpu.VMEM((K2,), jnp.int32),
        pltpu.VMEM((K2,), jnp.int32),
        pltpu.VMEM((K2,), jnp.float32),
        pltpu.VMEM((SEL,), jnp.int32),
        pltpu.VMEM((SEL,), jnp.int32),
        pltpu.VMEM((SEL,), jnp.float32),
        pltpu.VMEM((MAXB, 128), jnp.int32),
        pltpu.VMEM((MAXB, 128), jnp.int32),
        pltpu.VMEM((128, DD), jnp.float32),
        pltpu.VMEM_SHARED((RCH + 8, DD), jnp.float32),
        pltpu.SemaphoreType.DMA,
    ],
)
def _scatter_kernel(x_hbm, u_hbm, v_hbm, f_hbm, grad_hbm,
                    ub, vb, fb, dst1, src1, fac1, dst2, src2, rowbuf, acc,
                    sem):
    cid = lax.axis_index("c")
    sid = lax.axis_index("s")
    jv = jnp.where(lax.iota(jnp.int32, LL) == 0, -1.0, 1.0).astype(jnp.float32)
    row0 = sid * RPT

    def zrow(r, c):
        for k in range(DD // LL):
            rowbuf[r, pl.ds(k * LL, LL)] = jnp.zeros((LL,), jnp.float32)
        return c

    lax.fori_loop(0, 128, zrow, 0)

    def chunk(j, carry):
        c0 = (2 * j + cid) * RCH
        for q in range(RPT // 128):
            pltpu.sync_copy(rowbuf, acc.at[pl.ds(row0 + q * 128, 128)])
        pltpu.sync_copy(rowbuf.at[pl.ds(0, RPT % 128)],
                        acc.at[pl.ds(row0 + (RPT // 128) * 128, RPT % 128)])
        plsc.subcore_barrier()

        def block(jb, c2):
            base = (jb * NS + sid) * K2
            pltpu.sync_copy(u_hbm.at[pl.ds(base, K2)], ub)
            pltpu.sync_copy(v_hbm.at[pl.ds(base, K2)], vb)
            pltpu.sync_copy(f_hbm.at[pl.ds(base, K2)], fb)
            dummy = jnp.full((LL,), RCH, jnp.int32)
            zi = jnp.zeros((LL,), jnp.int32)
            zf = jnp.zeros((LL,), jnp.float32)
            for q in range(SEL // LL):
                dst1[pl.ds(q * LL, LL)] = dummy
                src1[pl.ds(q * LL, LL)] = zi
                fac1[pl.ds(q * LL, LL)] = zf

            def group(g, ptr):
                u16 = ub[pl.ds(g * LL, LL)]
                v16 = vb[pl.ds(g * LL, LL)]
                f16 = fb[pl.ds(g * LL, LL)]
                du = u16 - c0
                mu = (du >= 0) & (du < RCH)
                plsc.store_compressed(dst1.at[pl.ds(ptr, LL)], du, mask=mu)
                plsc.store_compressed(src1.at[pl.ds(ptr, LL)], v16, mask=mu)
                plsc.store_compressed(fac1.at[pl.ds(ptr, LL)], f16, mask=mu)
                ptr = ptr + plsc.all_reduce_population_count(mu)[0]
                dv = v16 - c0
                mv = (dv >= 0) & (dv < RCH)
                plsc.store_compressed(dst1.at[pl.ds(ptr, LL)], dv, mask=mv)
                plsc.store_compressed(src1.at[pl.ds(ptr, LL)], u16, mask=mv)
                plsc.store_compressed(fac1.at[pl.ds(ptr, LL)], f16, mask=mv)
                ptr = ptr + plsc.all_reduce_population_count(mv)[0]
                return ptr

            cnt = lax.fori_loop(0, K2 // LL, group, jnp.int32(0))
            nbat = (cnt + 127) // 128

            for b in range(MAXB):
                @pl.when(b < nbat)
                def _():
                    for k in range(128 // LL):
                        dst2[b, pl.ds(k * LL, LL)] = dst1[pl.ds(b * 128 + k * LL, LL)]
                        src2[b, pl.ds(k * LL, LL)] = src1[pl.ds(b * 128 + k * LL, LL)]
                    pltpu.async_copy(x_hbm.at[src2.at[b]], rowbuf, sem).wait()

                    def scale(g2, c3):
                        fv = fac1[pl.ds(b * 128 + g2 * LL, LL)]
                        for r in range(LL):
                            f = fv[r]
                            row = g2 * LL + r
                            rowbuf[row, pl.ds(0, LL)] = (
                                rowbuf[row, pl.ds(0, LL)] * (f * jv))
                            for k in range(1, DD // LL):
                                rowbuf[row, pl.ds(k * LL, LL)] = (
                                    rowbuf[row, pl.ds(k * LL, LL)] * f)
                        return c3

                    lax.fori_loop(0, 128 // LL, scale, 0)
                    pltpu.sync_copy(rowbuf, acc.at[dst2.at[b]], add=True)
            return c2

        lax.fori_loop(0, BPT2, block, 0)
        plsc.subcore_barrier()
        for q in range(RPT // 128):
            pltpu.sync_copy(acc.at[pl.ds(row0 + q * 128, 128)], rowbuf)
            pltpu.sync_copy(rowbuf, grad_hbm.at[pl.ds(c0 + row0 + q * 128, 128)])
        pltpu.sync_copy(acc.at[pl.ds(row0 + (RPT // 128) * 128, RPT % 128)],
                        rowbuf.at[pl.ds(0, RPT % 128)])
        pltpu.sync_copy(rowbuf.at[pl.ds(0, RPT % 128)],
                        grad_hbm.at[pl.ds(c0 + row0 + (RPT // 128) * 128, RPT % 128)])
        lax.fori_loop(0, 128, zrow, 0)
        plsc.subcore_barrier()
        return carry

    lax.fori_loop(0, NCH // NC, chunk, 0)


def kernel(x, u_idx, v_idx):
    pad = jnp.zeros((EP - EE,), jnp.int32)
    up = jnp.concatenate([u_idx, pad])
    vp = jnp.concatenate([v_idx, pad])
    inner = _inner_kernel(x, up, vp)
    factor2d, energy = _gate(inner.reshape(EPB, 128))
    grad_pad = _scatter_kernel(x, up, vp, factor2d.reshape(-1))
    return energy[0, 0], grad_pad[:NN]
